# speedup vs baseline: 1.1156x; 1.1156x over previous
import functools

import jax
import jax.numpy as jnp
from jax import lax
from jax.experimental import pallas as pl
from jax.experimental.pallas import tpu as pltpu
from jax.experimental.pallas import tpu_sc as plsc

_BATCH = 4096
_TEXT_DIM = 384
_OUT_DIM = 256
_NUM_CORES = 2
_NUM_SUBCORES = 16
_NW = _NUM_CORES * _NUM_SUBCORES
_B_PER_W = _BATCH // _NW
_TI = 512
_INV_TEMP = 1.0 / 0.07


def _sc_gather(item_table, ids):
    mesh = plsc.VectorSubcoreMesh(core_axis_name="c", subcore_axis_name="s")

    @functools.partial(
        pl.kernel,
        mesh=mesh,
        out_type=jax.ShapeDtypeStruct((_BATCH, _OUT_DIM), jnp.float32),
        scratch_types=[
            pltpu.VMEM((_B_PER_W,), jnp.int32),
            pltpu.VMEM((_B_PER_W, _OUT_DIM), jnp.float32),
            pltpu.SemaphoreType.DMA,
        ],
    )
    def k(table_hbm, idx_hbm, out_hbm, idx_v, rows_v, sem):
        wid = lax.axis_index("s") * _NUM_CORES + lax.axis_index("c")
        base = wid * _B_PER_W
        pltpu.sync_copy(idx_hbm.at[pl.ds(base, _B_PER_W)], idx_v)
        pltpu.async_copy(table_hbm.at[idx_v], rows_v, sem).wait()
        pltpu.sync_copy(rows_v, out_hbm.at[pl.ds(base, _B_PER_W)])

    return k(item_table, ids)


def _prep_body(text_ref, w_ref, b_ref, g_ref, be_ref, out_ref):
    h = jnp.dot(text_ref[...], w_ref[...], preferred_element_type=jnp.float32)
    h = h + b_ref[...]
    mu = jnp.mean(h, axis=-1, keepdims=True)
    var = jnp.mean((h - mu) ** 2, axis=-1, keepdims=True)
    h = (h - mu) * lax.rsqrt(var + 1e-5) * g_ref[...] + be_ref[...]
    n = jnp.sqrt(jnp.sum(h * h, axis=-1, keepdims=True))
    out_ref[...] = h * (_INV_TEMP / jnp.maximum(n, 1e-12))


def _mm_body(pref_ref, items_ref, out_ref, items_n):
    @pl.when(pl.program_id(0) == 0)
    def _():
        it = items_ref[...]
        n = jnp.sqrt(jnp.sum(it * it, axis=-1, keepdims=True))
        items_n[...] = it / jnp.maximum(n, 1e-12)

    out_ref[...] = lax.dot_general(
        pref_ref[...], items_n[...],
        (((1,), (1,)), ((), ())),
        preferred_element_type=jnp.float32,
    )


def kernel(text_embeddings, item_ids, W_proj, b_proj, ln_gamma, ln_beta,
           item_table):
    ids = item_ids.astype(jnp.int32)
    gathered = _sc_gather(item_table, ids)

    pref = pl.pallas_call(
        _prep_body,
        out_shape=jax.ShapeDtypeStruct((_BATCH, _OUT_DIM), jnp.float32),
    )(
        text_embeddings,
        W_proj,
        b_proj.reshape(1, _OUT_DIM),
        ln_gamma.reshape(1, _OUT_DIM),
        ln_beta.reshape(1, _OUT_DIM),
    )

    logits = pl.pallas_call(
        _mm_body,
        grid=(_BATCH // _TI,),
        in_specs=[
            pl.BlockSpec((_TI, _OUT_DIM), lambda i: (i, 0)),
            pl.BlockSpec((_BATCH, _OUT_DIM), lambda i: (0, 0)),
        ],
        out_specs=pl.BlockSpec((_TI, _BATCH), lambda i: (i, 0)),
        out_shape=jax.ShapeDtypeStruct((_BATCH, _BATCH), jnp.float32),
        scratch_shapes=[pltpu.VMEM((_BATCH, _OUT_DIM), jnp.float32)],
    )(pref, gathered)

    return logits


# speedup vs baseline: 1.1388x; 1.0208x over previous
import functools

import jax
import jax.numpy as jnp
from jax import lax
from jax.experimental import pallas as pl
from jax.experimental.pallas import tpu as pltpu
from jax.experimental.pallas import tpu_sc as plsc

_BATCH = 4096
_TEXT_DIM = 384
_OUT_DIM = 256
_NUM_CORES = 2
_NUM_SUBCORES = 16
_NW = _NUM_CORES * _NUM_SUBCORES
_B_PER_W = _BATCH // _NW
_TI = 512
_INV_TEMP = 1.0 / 0.07


def _sc_gather(item_table, ids):
    mesh = plsc.VectorSubcoreMesh(core_axis_name="c", subcore_axis_name="s")

    @functools.partial(
        pl.kernel,
        mesh=mesh,
        out_type=jax.ShapeDtypeStruct((_BATCH, _OUT_DIM), jnp.float32),
        scratch_types=[
            pltpu.VMEM((_B_PER_W,), jnp.int32),
            pltpu.VMEM((_B_PER_W, _OUT_DIM), jnp.float32),
            pltpu.SemaphoreType.DMA,
        ],
    )
    def k(table_hbm, idx_hbm, out_hbm, idx_v, rows_v, sem):
        wid = lax.axis_index("s") * _NUM_CORES + lax.axis_index("c")
        base = wid * _B_PER_W
        pltpu.sync_copy(idx_hbm.at[pl.ds(base, _B_PER_W)], idx_v)
        pltpu.async_copy(table_hbm.at[idx_v], rows_v, sem).wait()
        pltpu.sync_copy(rows_v, out_hbm.at[pl.ds(base, _B_PER_W)])

    return k(item_table, ids)


def _prep_body(text_ref, w_ref, b_ref, g_ref, be_ref, out_ref):
    h = jnp.dot(text_ref[...], w_ref[...], preferred_element_type=jnp.float32)
    h = h + b_ref[...]
    mu = jnp.mean(h, axis=-1, keepdims=True)
    var = jnp.mean((h - mu) ** 2, axis=-1, keepdims=True)
    h = (h - mu) * lax.rsqrt(var + 1e-5) * g_ref[...] + be_ref[...]
    n = jnp.sqrt(jnp.sum(h * h, axis=-1, keepdims=True))
    out_ref[...] = (h * (_INV_TEMP / jnp.maximum(n, 1e-12))).astype(jnp.bfloat16)


def _mm_body(pref_ref, items_ref, out_ref, items_n):
    @pl.when(pl.program_id(0) == 0)
    def _():
        it = items_ref[...]
        n = jnp.sqrt(jnp.sum(it * it, axis=-1, keepdims=True))
        items_n[...] = (it / jnp.maximum(n, 1e-12)).astype(jnp.bfloat16)

    out_ref[...] = lax.dot_general(
        pref_ref[...], items_n[...],
        (((1,), (1,)), ((), ())),
        preferred_element_type=jnp.float32,
    )


def kernel(text_embeddings, item_ids, W_proj, b_proj, ln_gamma, ln_beta,
           item_table):
    ids = item_ids.astype(jnp.int32)
    gathered = _sc_gather(item_table, ids)

    pref = pl.pallas_call(
        _prep_body,
        out_shape=jax.ShapeDtypeStruct((_BATCH, _OUT_DIM), jnp.bfloat16),
    )(
        text_embeddings,
        W_proj,
        b_proj.reshape(1, _OUT_DIM),
        ln_gamma.reshape(1, _OUT_DIM),
        ln_beta.reshape(1, _OUT_DIM),
    )

    logits = pl.pallas_call(
        _mm_body,
        grid=(_BATCH // _TI,),
        in_specs=[
            pl.BlockSpec((_TI, _OUT_DIM), lambda i: (i, 0)),
            pl.BlockSpec((_BATCH, _OUT_DIM), lambda i: (0, 0)),
        ],
        out_specs=pl.BlockSpec((_TI, _BATCH), lambda i: (i, 0)),
        out_shape=jax.ShapeDtypeStruct((_BATCH, _BATCH), jnp.float32),
        scratch_shapes=[pltpu.VMEM((_BATCH, _OUT_DIM), jnp.bfloat16)],
    )(pref, gathered)

    return logits
